# baseline (device time: 51039 ns/iter reference)
import jax
import jax.numpy as jnp
from jax import lax
from jax.experimental import pallas as pl
from jax.experimental.pallas import tpu as pltpu

N_DEV = 4
B_LOC = 2
SQ = 256
SKV = 256
HQ = 16
HG = 4
DH = 64
D_MODEL = 512
GROUP = HG * DH


def kernel(x, Wq, K_ext, V_ext, Wo):
    my = lax.axis_index("i")
    k_loc = lax.dynamic_slice_in_dim(K_ext, my * B_LOC, B_LOC, axis=0)
    v_loc = lax.dynamic_slice_in_dim(V_ext, my * B_LOC, B_LOC, axis=0)
    k_t = jnp.transpose(k_loc, (2, 0, 1, 3))
    v_t = jnp.transpose(v_loc, (2, 0, 1, 3))

    def body(x_ref, wq_ref, wo_ref, k_ref, v_ref, out_ref,
             wq_buf, wo_buf, bias_ref,
             wq_send, wq_recv, wo_send, wo_recv):
        my_pos = lax.axis_index("i")
        right = lax.rem(my_pos + 1, N_DEV)
        left = lax.rem(my_pos + N_DEV - 1, N_DEV)

        barrier = pltpu.get_barrier_semaphore()
        for nbr in (left, right):
            pl.semaphore_signal(barrier, inc=1, device_id=(nbr,),
                                device_id_type=pl.DeviceIdType.MESH)
        pl.semaphore_wait(barrier, 2)

        wq_buf[0] = wq_ref[...]
        wo_buf[0] = wo_ref[...]

        qi = lax.broadcasted_iota(jnp.int32, (SQ, SKV), 0)
        ki = lax.broadcasted_iota(jnp.int32, (SQ, SKV), 1)
        mask = (jnp.abs(qi - ki) <= 128) | (ki < 32) | (qi < 32)
        bias_ref[...] = jnp.where(mask, 0.0, -1e9)

        for h in range(N_DEV):
            if h < N_DEV - 1:
                rq = pltpu.make_async_remote_copy(
                    src_ref=wq_buf.at[h], dst_ref=wq_buf.at[h + 1],
                    send_sem=wq_send.at[h], recv_sem=wq_recv.at[h],
                    device_id=(right,), device_id_type=pl.DeviceIdType.MESH)
                ro = pltpu.make_async_remote_copy(
                    src_ref=wo_buf.at[h], dst_ref=wo_buf.at[h + 1],
                    send_sem=wo_send.at[h], recv_sem=wo_recv.at[h],
                    device_id=(right,), device_id_type=pl.DeviceIdType.MESH)
                rq.start()
                ro.start()

            origin = lax.rem(my_pos - h + N_DEV, N_DEV)
            for b in range(B_LOC):
                qb = jnp.dot(x_ref[b], wq_buf[h],
                             preferred_element_type=jnp.float32)
                ctx = []
                for hh in range(HG):
                    head = origin * HG + hh
                    kh = k_ref[head, b]
                    vh = v_ref[head, b]
                    q = qb[:, hh * DH:(hh + 1) * DH]
                    s = lax.dot_general(
                        q, kh, (((1,), (1,)), ((), ())),
                        preferred_element_type=jnp.float32) * 0.125
                    s = s + bias_ref[...]
                    m = jnp.max(s, axis=-1, keepdims=True)
                    w = jnp.exp(s - m)
                    w = w / jnp.sum(w, axis=-1, keepdims=True)
                    ctx.append(jnp.dot(w, vh,
                                       preferred_element_type=jnp.float32))
                ctx = jnp.concatenate(ctx, axis=1)
                contrib = jnp.dot(ctx, wo_buf[h],
                                  preferred_element_type=jnp.float32)
                if h == 0:
                    out_ref[b] = contrib
                else:
                    out_ref[b] = out_ref[b] + contrib

            if h < N_DEV - 1:
                rq.wait()
                ro.wait()

    return pl.pallas_call(
        body,
        out_shape=jax.ShapeDtypeStruct((B_LOC, SQ, D_MODEL), jnp.float32),
        in_specs=[pl.BlockSpec(memory_space=pltpu.VMEM)] * 5,
        out_specs=pl.BlockSpec(memory_space=pltpu.VMEM),
        scratch_shapes=[
            pltpu.VMEM((N_DEV, D_MODEL, GROUP), jnp.float32),
            pltpu.VMEM((N_DEV, GROUP, D_MODEL), jnp.float32),
            pltpu.VMEM((SQ, SKV), jnp.float32),
            pltpu.SemaphoreType.DMA((N_DEV - 1,)),
            pltpu.SemaphoreType.DMA((N_DEV - 1,)),
            pltpu.SemaphoreType.DMA((N_DEV - 1,)),
            pltpu.SemaphoreType.DMA((N_DEV - 1,)),
        ],
        compiler_params=pltpu.CompilerParams(collective_id=0),
    )(x, Wq, Wo, k_t, v_t)


# device time: 34373 ns/iter; 1.4849x vs baseline; 1.4849x over previous
import jax
import jax.numpy as jnp
from jax import lax
from jax.experimental import pallas as pl
from jax.experimental.pallas import tpu as pltpu

N_DEV = 4
B_LOC = 2
SQ = 256
SKV = 256
HQ = 16
HG = 4
DH = 64
D_MODEL = 512
GROUP = HG * DH


def kernel(x, Wq, K_ext, V_ext, Wo):
    my = lax.axis_index("i")
    k_loc = lax.dynamic_slice_in_dim(K_ext, my * B_LOC, B_LOC, axis=0)
    v_loc = lax.dynamic_slice_in_dim(V_ext, my * B_LOC, B_LOC, axis=0)
    k_t = jnp.transpose(k_loc, (2, 0, 1, 3))
    v_t = jnp.transpose(v_loc, (2, 0, 1, 3))

    def body(x_ref, wq_ref, wo_ref, k_ref, v_ref, out_ref,
             wq_l, wo_l, wq_r, wo_r, wq_o, wo_o, bias_ref,
             send_sems, recv_sems):
        my_pos = lax.axis_index("i")
        right = lax.rem(my_pos + 1, N_DEV)
        left = lax.rem(my_pos + N_DEV - 1, N_DEV)
        opp = lax.rem(my_pos + 2, N_DEV)

        barrier = pltpu.get_barrier_semaphore()
        for nbr in (left, right):
            pl.semaphore_signal(barrier, inc=1, device_id=(nbr,),
                                device_id_type=pl.DeviceIdType.MESH)
        pl.semaphore_wait(barrier, 2)

        qi = lax.broadcasted_iota(jnp.int32, (SQ, SKV), 0)
        ki = lax.broadcasted_iota(jnp.int32, (SQ, SKV), 1)
        mask = (jnp.abs(qi - ki) <= 128) | (ki < 32) | (qi < 32)
        bias_ref[...] = jnp.where(mask, 0.0, -1e9)

        def rdma(src, dst, i, dev):
            return pltpu.make_async_remote_copy(
                src_ref=src, dst_ref=dst,
                send_sem=send_sems.at[i], recv_sem=recv_sems.at[i],
                device_id=(dev,), device_id_type=pl.DeviceIdType.MESH)

        def compute(wq_s, wo_s, origin, first):
            for b in range(B_LOC):
                qb = jnp.dot(x_ref[b], wq_s[...],
                             preferred_element_type=jnp.float32)
                ctx = []
                for hh in range(HG):
                    head = origin * HG + hh
                    kh = k_ref[head, b]
                    vh = v_ref[head, b]
                    q = qb[:, hh * DH:(hh + 1) * DH]
                    s = lax.dot_general(
                        q, kh, (((1,), (1,)), ((), ())),
                        preferred_element_type=jnp.float32) * 0.125
                    s = s + bias_ref[...]
                    m = jnp.max(s, axis=-1, keepdims=True)
                    w = jnp.exp(s - m)
                    w = w / jnp.sum(w, axis=-1, keepdims=True)
                    ctx.append(jnp.dot(w, vh,
                                       preferred_element_type=jnp.float32))
                ctx = jnp.concatenate(ctx, axis=1)
                contrib = jnp.dot(ctx, wo_s[...],
                                  preferred_element_type=jnp.float32)
                if first:
                    out_ref[b] = contrib
                else:
                    out_ref[b] = out_ref[b] + contrib

        t0 = rdma(wq_ref, wq_l, 0, right)
        t1 = rdma(wo_ref, wo_l, 1, right)
        t2 = rdma(wq_ref, wq_r, 2, left)
        t3 = rdma(wo_ref, wo_r, 3, left)
        for t in (t0, t1, t2, t3):
            t.start()

        compute(wq_ref, wo_ref, my_pos, first=True)

        t0.wait_recv()
        t4 = rdma(wq_l, wq_o, 4, right)
        t4.start()
        t1.wait_recv()
        compute(wq_l, wo_l, left, first=False)

        t3.wait_recv()
        t5 = rdma(wo_r, wo_o, 5, left)
        t5.start()
        t2.wait_recv()
        compute(wq_r, wo_r, right, first=False)

        t4.wait_recv()
        t5.wait_recv()
        compute(wq_o, wo_o, opp, first=False)

        for t in (t0, t1, t2, t3, t4, t5):
            t.wait_send()

    return pl.pallas_call(
        body,
        out_shape=jax.ShapeDtypeStruct((B_LOC, SQ, D_MODEL), jnp.float32),
        in_specs=[pl.BlockSpec(memory_space=pltpu.VMEM)] * 5,
        out_specs=pl.BlockSpec(memory_space=pltpu.VMEM),
        scratch_shapes=[
            pltpu.VMEM((D_MODEL, GROUP), jnp.float32),
            pltpu.VMEM((GROUP, D_MODEL), jnp.float32),
            pltpu.VMEM((D_MODEL, GROUP), jnp.float32),
            pltpu.VMEM((GROUP, D_MODEL), jnp.float32),
            pltpu.VMEM((D_MODEL, GROUP), jnp.float32),
            pltpu.VMEM((GROUP, D_MODEL), jnp.float32),
            pltpu.VMEM((SQ, SKV), jnp.float32),
            pltpu.SemaphoreType.DMA((6,)),
            pltpu.SemaphoreType.DMA((6,)),
        ],
        compiler_params=pltpu.CompilerParams(collective_id=0),
    )(x, Wq, Wo, k_t, v_t)


# device time: 26005 ns/iter; 1.9627x vs baseline; 1.3218x over previous
import jax
import jax.numpy as jnp
from jax import lax
from jax.experimental import pallas as pl
from jax.experimental.pallas import tpu as pltpu

N_DEV = 4
B_LOC = 2
SQ = 256
SKV = 256
HQ = 16
HG = 4
DH = 64
D_MODEL = 512
GROUP = HG * DH


def kernel(x, Wq, K_ext, V_ext, Wo):
    my = lax.axis_index("i")
    k_loc = lax.dynamic_slice_in_dim(K_ext, my * B_LOC, B_LOC, axis=0)
    v_loc = lax.dynamic_slice_in_dim(V_ext, my * B_LOC, B_LOC, axis=0)
    k_t = jnp.transpose(k_loc, (2, 0, 1, 3))
    v_t = jnp.transpose(v_loc, (2, 0, 1, 3))
    x16 = x.astype(jnp.bfloat16)
    wq16 = Wq.astype(jnp.bfloat16)
    wo16 = Wo.astype(jnp.bfloat16)

    def body(x_ref, wq_ref, wo_ref, k_ref, v_ref, out_ref,
             wq_l, wo_l, wq_r, wo_r, wq_o, wo_o, bias_ref,
             send_sems, recv_sems):
        my_pos = lax.axis_index("i")
        right = lax.rem(my_pos + 1, N_DEV)
        left = lax.rem(my_pos + N_DEV - 1, N_DEV)
        opp = lax.rem(my_pos + 2, N_DEV)

        barrier = pltpu.get_barrier_semaphore()
        for nbr in (left, right):
            pl.semaphore_signal(barrier, inc=1, device_id=(nbr,),
                                device_id_type=pl.DeviceIdType.MESH)
        pl.semaphore_wait(barrier, 2)

        qi = lax.broadcasted_iota(jnp.int32, (SQ, SKV), 0)
        ki = lax.broadcasted_iota(jnp.int32, (SQ, SKV), 1)
        mask = (jnp.abs(qi - ki) <= 128) | (ki < 32) | (qi < 32)
        bias_ref[...] = jnp.where(mask, 0.0, -1e9)

        def rdma(src, dst, i, dev):
            return pltpu.make_async_remote_copy(
                src_ref=src, dst_ref=dst,
                send_sem=send_sems.at[i], recv_sem=recv_sems.at[i],
                device_id=(dev,), device_id_type=pl.DeviceIdType.MESH)

        def compute(wq_s, wo_s, origin, first):
            for b in range(B_LOC):
                qb = jnp.dot(x_ref[b], wq_s[...],
                             preferred_element_type=jnp.float32)
                ctx = []
                for hh in range(HG):
                    head = origin * HG + hh
                    kh = k_ref[head, b]
                    vh = v_ref[head, b]
                    q = qb[:, hh * DH:(hh + 1) * DH]
                    s = lax.dot_general(
                        q, kh, (((1,), (1,)), ((), ())),
                        preferred_element_type=jnp.float32) * 0.125
                    s = s + bias_ref[...]
                    m = jnp.max(s, axis=-1, keepdims=True)
                    w = jnp.exp(s - m)
                    w = w / jnp.sum(w, axis=-1, keepdims=True)
                    ctx.append(jnp.dot(w, vh,
                                       preferred_element_type=jnp.float32))
                ctx = jnp.concatenate(ctx, axis=1)
                contrib = jnp.dot(ctx.astype(jnp.bfloat16), wo_s[...],
                                  preferred_element_type=jnp.float32)
                if first:
                    out_ref[b] = contrib
                else:
                    out_ref[b] = out_ref[b] + contrib

        t0 = rdma(wq_ref, wq_l, 0, right)
        t1 = rdma(wo_ref, wo_l, 1, right)
        t2 = rdma(wq_ref, wq_r, 2, left)
        t3 = rdma(wo_ref, wo_r, 3, left)
        for t in (t0, t1, t2, t3):
            t.start()

        compute(wq_ref, wo_ref, my_pos, first=True)

        t0.wait_recv()
        t4 = rdma(wq_l, wq_o, 4, right)
        t4.start()
        t1.wait_recv()
        compute(wq_l, wo_l, left, first=False)

        t3.wait_recv()
        t5 = rdma(wo_r, wo_o, 5, left)
        t5.start()
        t2.wait_recv()
        compute(wq_r, wo_r, right, first=False)

        t4.wait_recv()
        t5.wait_recv()
        compute(wq_o, wo_o, opp, first=False)

        for t in (t0, t1, t2, t3, t4, t5):
            t.wait_send()

    return pl.pallas_call(
        body,
        out_shape=jax.ShapeDtypeStruct((B_LOC, SQ, D_MODEL), jnp.float32),
        in_specs=[pl.BlockSpec(memory_space=pltpu.VMEM)] * 5,
        out_specs=pl.BlockSpec(memory_space=pltpu.VMEM),
        scratch_shapes=[
            pltpu.VMEM((D_MODEL, GROUP), jnp.bfloat16),
            pltpu.VMEM((GROUP, D_MODEL), jnp.bfloat16),
            pltpu.VMEM((D_MODEL, GROUP), jnp.bfloat16),
            pltpu.VMEM((GROUP, D_MODEL), jnp.bfloat16),
            pltpu.VMEM((D_MODEL, GROUP), jnp.bfloat16),
            pltpu.VMEM((GROUP, D_MODEL), jnp.bfloat16),
            pltpu.VMEM((SQ, SKV), jnp.float32),
            pltpu.SemaphoreType.DMA((6,)),
            pltpu.SemaphoreType.DMA((6,)),
        ],
        compiler_params=pltpu.CompilerParams(collective_id=0),
    )(x16, wq16, wo16, k_t, v_t)
